# Initial kernel scaffold; baseline (speedup 1.0000x reference)
import jax
import jax.numpy as jnp
from jax.experimental import pallas as pl


def kernel(x, edge_index, W1, b1, Wh, bh, W2, b2):
    raise NotImplementedError("write your pallas kernel here")


# speedup vs baseline: 18.0915x; 18.0915x over previous
import functools

import jax
import jax.numpy as jnp
from jax import lax
from jax.experimental import pallas as pl
from jax.experimental.pallas import tpu as pltpu
from jax.experimental.pallas import tpu_sc as plsc

N = 10000
E = 320000
F = 128

NC = 2
NS = 16
NW = NC * NS
EB = 80
EROWS = E // EB
BLK_PER_TILE = EROWS // NW
NCHUNK = 5
CB = BLK_PER_TILE // NCHUNK
NPT = 624
NTAIL_OFF = NPT * NS
NTAIL = N - NTAIL_OFF

NPAD = 10240
DSEG = NPAD // NS
DEGW = 128

_mesh = plsc.VectorSubcoreMesh(
    core_axis_name="c", subcore_axis_name="s",
    num_cores=NC, num_subcores=NS)


@functools.partial(
    pl.kernel,
    out_type=jax.ShapeDtypeStruct((NC, NPAD, DEGW), jnp.float32),
    mesh=_mesh,
    scratch_types=[
        pltpu.VMEM((BLK_PER_TILE, EB), jnp.int32),
        pltpu.VMEM((EB, DEGW), jnp.float32),
        pltpu.VMEM_SHARED((NPAD, DEGW), jnp.float32),
    ],
)
def _sc_degree(dst_hbm, zeros_hbm, ones_hbm, out_hbm, dst_v, ones_v, deg_sh):
    c = lax.axis_index("c")
    s = lax.axis_index("s")
    wid = s * NC + c
    pltpu.sync_copy(zeros_hbm.at[pl.ds(s * DSEG, DSEG)],
                    deg_sh.at[pl.ds(s * DSEG, DSEG)])
    pltpu.sync_copy(ones_hbm, ones_v)
    pltpu.sync_copy(dst_hbm.at[wid], dst_v)
    plsc.subcore_barrier()

    def step(j, carry):
        pltpu.sync_copy(ones_v, deg_sh.at[dst_v.at[j]], add=True)
        return carry

    lax.fori_loop(0, BLK_PER_TILE, step, 0)
    plsc.subcore_barrier()
    pltpu.sync_copy(deg_sh.at[pl.ds(s * DSEG, DSEG)],
                    out_hbm.at[c, pl.ds(s * DSEG, DSEG)])


@functools.partial(
    pl.kernel,
    out_type=jax.ShapeDtypeStruct((NC, N, F), jnp.float32),
    mesh=_mesh,
    scratch_types=[
        pltpu.VMEM((CB, EB), jnp.int32),
        pltpu.VMEM((CB, EB), jnp.int32),
        pltpu.VMEM((2, EB, F), jnp.float32),
        pltpu.VMEM_SHARED((N, F), jnp.float32),
        pltpu.SemaphoreType.DMA((2,)),
    ],
)
def _sc_aggregate(y_hbm, src_hbm, dst_hbm, out_hbm,
                  src_v, dst_v, rows_v, z_sh, sem):
    c = lax.axis_index("c")
    s = lax.axis_index("s")
    wid = s * NC + c
    pltpu.sync_copy(y_hbm.at[pl.ds(s * NPT, NPT)],
                    z_sh.at[pl.ds(s * NPT, NPT)])

    @pl.when(s == 0)
    def _init_tail():
        pltpu.sync_copy(y_hbm.at[pl.ds(NTAIL_OFF, NTAIL)],
                        z_sh.at[pl.ds(NTAIL_OFF, NTAIL)])

    pltpu.sync_copy(src_hbm.at[wid, 0], src_v)
    pltpu.sync_copy(dst_hbm.at[wid, 0], dst_v)
    plsc.subcore_barrier()

    pltpu.async_copy(y_hbm.at[src_v.at[0]], rows_v.at[0], sem.at[0])

    def step(j, carry):
        b = lax.rem(j, 2)
        r = lax.rem(j, CB)
        k = lax.div(j, CB)
        pltpu.make_async_copy(y_hbm.at[src_v.at[r]], rows_v.at[b],
                              sem.at[b]).wait()

        @pl.when(r + 1 < CB)
        def _prefetch():
            pltpu.async_copy(y_hbm.at[src_v.at[r + 1]], rows_v.at[1 - b],
                             sem.at[1 - b])

        pltpu.sync_copy(rows_v.at[b], z_sh.at[dst_v.at[r]], add=True)

        @pl.when(jnp.logical_and(r + 1 == CB, j + 1 < BLK_PER_TILE))
        def _next_chunk():
            pltpu.sync_copy(src_hbm.at[wid, k + 1], src_v)
            pltpu.sync_copy(dst_hbm.at[wid, k + 1], dst_v)
            pltpu.async_copy(y_hbm.at[src_v.at[0]], rows_v.at[1 - b],
                             sem.at[1 - b])
        return carry

    lax.fori_loop(0, BLK_PER_TILE, step, 0)
    plsc.subcore_barrier()
    pltpu.sync_copy(z_sh.at[pl.ds(s * NPT, NPT)],
                    out_hbm.at[c, pl.ds(s * NPT, NPT)])

    @pl.when(s == 0)
    def _out_tail():
        pltpu.sync_copy(z_sh.at[pl.ds(NTAIL_OFF, NTAIL)],
                        out_hbm.at[c, pl.ds(NTAIL_OFF, NTAIL)])


_RB = 1000
_GRID = N // _RB


def _tc_first_body(x_ref, da_ref, db_ref, w_ref, y_ref, dinv_ref):
    deg = da_ref[...] + db_ref[...] + 1.0
    dinv = lax.rsqrt(deg)
    y_ref[...] = jnp.dot(x_ref[...], w_ref[...],
                         preferred_element_type=jnp.float32) * dinv
    dinv_ref[...] = dinv


_tc_first = pl.pallas_call(
    _tc_first_body,
    grid=(_GRID,),
    in_specs=[
        pl.BlockSpec((_RB, F), lambda i: (i, 0)),
        pl.BlockSpec((_RB, 1), lambda i: (i, 0)),
        pl.BlockSpec((_RB, 1), lambda i: (i, 0)),
        pl.BlockSpec((F, F), lambda i: (0, 0)),
    ],
    out_specs=[
        pl.BlockSpec((_RB, F), lambda i: (i, 0)),
        pl.BlockSpec((_RB, 1), lambda i: (i, 0)),
    ],
    out_shape=[
        jax.ShapeDtypeStruct((N, F), jnp.float32),
        jax.ShapeDtypeStruct((N, 1), jnp.float32),
    ],
)


def _tc_mid_body(z0_ref, z1_ref, yp_ref, dinv_ref, b_ref, w_ref, y_ref):
    dinv = dinv_ref[...]
    h = (z0_ref[...] + z1_ref[...] - yp_ref[...]) * dinv + b_ref[...]
    h = jnp.maximum(h, 0.0)
    y_ref[...] = jnp.dot(h, w_ref[...],
                         preferred_element_type=jnp.float32) * dinv


_tc_mid = pl.pallas_call(
    _tc_mid_body,
    grid=(_GRID,),
    in_specs=[
        pl.BlockSpec((_RB, F), lambda i: (i, 0)),
        pl.BlockSpec((_RB, F), lambda i: (i, 0)),
        pl.BlockSpec((_RB, F), lambda i: (i, 0)),
        pl.BlockSpec((_RB, 1), lambda i: (i, 0)),
        pl.BlockSpec((1, F), lambda i: (0, 0)),
        pl.BlockSpec((F, F), lambda i: (0, 0)),
    ],
    out_specs=pl.BlockSpec((_RB, F), lambda i: (i, 0)),
    out_shape=jax.ShapeDtypeStruct((N, F), jnp.float32),
)


def _tc_final_body(z0_ref, z1_ref, yp_ref, dinv_ref, b_ref, out_ref):
    out_ref[...] = ((z0_ref[...] + z1_ref[...] - yp_ref[...])
                    * dinv_ref[...] + b_ref[...])


_tc_final = pl.pallas_call(
    _tc_final_body,
    grid=(_GRID,),
    in_specs=[
        pl.BlockSpec((_RB, F), lambda i: (i, 0)),
        pl.BlockSpec((_RB, F), lambda i: (i, 0)),
        pl.BlockSpec((_RB, F), lambda i: (i, 0)),
        pl.BlockSpec((_RB, 1), lambda i: (i, 0)),
        pl.BlockSpec((1, F), lambda i: (0, 0)),
    ],
    out_specs=pl.BlockSpec((_RB, F), lambda i: (i, 0)),
    out_shape=jax.ShapeDtypeStruct((N, F), jnp.float32),
)


def kernel(x, edge_index, W1, b1, Wh, bh, W2, b2):
    edge_index = edge_index.astype(jnp.int32)
    src4 = edge_index[0].reshape(NW, NCHUNK, CB, EB)
    dst4 = edge_index[1].reshape(NW, NCHUNK, CB, EB)
    dst3 = edge_index[1].reshape(NW, BLK_PER_TILE, EB)

    zeros_pad = jnp.zeros((NPAD, DEGW), jnp.float32)
    ones_blk = jnp.ones((EB, DEGW), jnp.float32)
    deg2 = _sc_degree(dst3, zeros_pad, ones_blk)
    deg_a = deg2[0, :N, 0].reshape(N, 1)
    deg_b = deg2[1, :N, 0].reshape(N, 1)

    y1, dinv = _tc_first(x, deg_a, deg_b, W1)
    z = _sc_aggregate(y1, src4, dst4)
    y2 = _tc_mid(z[0], z[1], y1, dinv, b1.reshape(1, F), Wh)
    z = _sc_aggregate(y2, src4, dst4)
    y3 = _tc_mid(z[0], z[1], y2, dinv, bh.reshape(1, F), W2)
    z = _sc_aggregate(y3, src4, dst4)
    return _tc_final(z[0], z[1], y3, dinv, b2.reshape(1, F))


# speedup vs baseline: 18.4310x; 1.0188x over previous
import functools

import jax
import jax.numpy as jnp
from jax import lax
from jax.experimental import pallas as pl
from jax.experimental.pallas import tpu as pltpu
from jax.experimental.pallas import tpu_sc as plsc

N = 10000
E = 320000
F = 128

NC = 2
NS = 16
NW = NC * NS
EB = 80
EROWS = E // EB
BLK_PER_TILE = EROWS // NW
NCHUNK = 5
CB = BLK_PER_TILE // NCHUNK
NPT = 624
NTAIL_OFF = NPT * NS
NTAIL = N - NTAIL_OFF

NPAD = 10240
DSEG = NPAD // NS
DEGW = 128

_mesh = plsc.VectorSubcoreMesh(
    core_axis_name="c", subcore_axis_name="s",
    num_cores=NC, num_subcores=NS)


@functools.partial(
    pl.kernel,
    out_type=jax.ShapeDtypeStruct((NC, NPAD, DEGW), jnp.float32),
    mesh=_mesh,
    scratch_types=[
        pltpu.VMEM((BLK_PER_TILE, EB), jnp.int32),
        pltpu.VMEM((EB, DEGW), jnp.float32),
        pltpu.VMEM_SHARED((NPAD, DEGW), jnp.float32),
    ],
)
def _sc_degree(dst_hbm, zeros_hbm, ones_hbm, out_hbm, dst_v, ones_v, deg_sh):
    c = lax.axis_index("c")
    s = lax.axis_index("s")
    wid = s * NC + c
    pltpu.sync_copy(zeros_hbm.at[pl.ds(s * DSEG, DSEG)],
                    deg_sh.at[pl.ds(s * DSEG, DSEG)])
    pltpu.sync_copy(ones_hbm, ones_v)
    pltpu.sync_copy(dst_hbm.at[wid], dst_v)
    plsc.subcore_barrier()

    def step(j, carry):
        pltpu.sync_copy(ones_v, deg_sh.at[dst_v.at[j]], add=True)
        return carry

    lax.fori_loop(0, BLK_PER_TILE, step, 0)
    plsc.subcore_barrier()
    pltpu.sync_copy(deg_sh.at[pl.ds(s * DSEG, DSEG)],
                    out_hbm.at[c, pl.ds(s * DSEG, DSEG)])


@functools.partial(
    pl.kernel,
    out_type=jax.ShapeDtypeStruct((NC, N, F), jnp.float32),
    mesh=_mesh,
    scratch_types=[
        pltpu.VMEM((2, CB, EB), jnp.int32),
        pltpu.VMEM((2, CB, EB), jnp.int32),
        pltpu.VMEM((2, EB, F), jnp.float32),
        pltpu.VMEM_SHARED((N, F), jnp.float32),
        pltpu.SemaphoreType.DMA((2,)),
        pltpu.SemaphoreType.DMA((2,)),
        pltpu.SemaphoreType.DMA((2,)),
    ],
)
def _sc_aggregate(y_hbm, src_hbm, dst_hbm, out_hbm,
                  src_v, dst_v, rows_v, z_sh, gsem, ssem, isem):
    c = lax.axis_index("c")
    s = lax.axis_index("s")
    wid = s * NC + c
    pltpu.sync_copy(y_hbm.at[pl.ds(s * NPT, NPT)],
                    z_sh.at[pl.ds(s * NPT, NPT)])

    @pl.when(s == 0)
    def _init_tail():
        pltpu.sync_copy(y_hbm.at[pl.ds(NTAIL_OFF, NTAIL)],
                        z_sh.at[pl.ds(NTAIL_OFF, NTAIL)])

    pltpu.sync_copy(src_hbm.at[wid, 0], src_v.at[0])
    pltpu.sync_copy(dst_hbm.at[wid, 0], dst_v.at[0])
    plsc.subcore_barrier()

    pltpu.async_copy(y_hbm.at[src_v.at[0, 0]], rows_v.at[0], gsem.at[0])

    def step(j, carry):
        b = lax.rem(j, 2)
        r = lax.rem(j, CB)
        kp = lax.rem(lax.div(j, CB), 2)
        pltpu.make_async_copy(y_hbm.at[src_v.at[kp, r]], rows_v.at[b],
                              gsem.at[b]).wait()
        pltpu.async_copy(rows_v.at[b], z_sh.at[dst_v.at[kp, r]],
                         ssem.at[b], add=True)

        @pl.when(j > 0)
        def _drain():
            pltpu.make_async_copy(rows_v.at[1 - b], z_sh.at[dst_v.at[kp, r]],
                                  ssem.at[1 - b]).wait()

        @pl.when(jnp.logical_and(r == 0, j + CB < BLK_PER_TILE))
        def _stage_chunk():
            k = lax.div(j, CB)
            pltpu.async_copy(src_hbm.at[wid, k + 1], src_v.at[1 - kp],
                             isem.at[0])
            pltpu.async_copy(dst_hbm.at[wid, k + 1], dst_v.at[1 - kp],
                             isem.at[1])

        jn = j + 1

        @pl.when(jn < BLK_PER_TILE)
        def _prefetch():
            rn = lax.rem(jn, CB)
            kpn = lax.rem(lax.div(jn, CB), 2)

            @pl.when(rn == 0)
            def _wait_chunk():
                pltpu.make_async_copy(src_hbm.at[wid, 0], src_v.at[0],
                                      isem.at[0]).wait()
                pltpu.make_async_copy(dst_hbm.at[wid, 0], dst_v.at[0],
                                      isem.at[1]).wait()

            pltpu.async_copy(y_hbm.at[src_v.at[kpn, rn]], rows_v.at[1 - b],
                             gsem.at[1 - b])
        return carry

    lax.fori_loop(0, BLK_PER_TILE, step, 0)
    pltpu.make_async_copy(rows_v.at[0], z_sh.at[dst_v.at[0, 0]],
                          ssem.at[0]).wait()
    plsc.subcore_barrier()
    pltpu.sync_copy(z_sh.at[pl.ds(s * NPT, NPT)],
                    out_hbm.at[c, pl.ds(s * NPT, NPT)])

    @pl.when(s == 0)
    def _out_tail():
        pltpu.sync_copy(z_sh.at[pl.ds(NTAIL_OFF, NTAIL)],
                        out_hbm.at[c, pl.ds(NTAIL_OFF, NTAIL)])


_RB = 1000
_GRID = N // _RB


def _tc_first_body(x_ref, da_ref, db_ref, w_ref, y_ref, dinv_ref):
    deg = da_ref[...] + db_ref[...] + 1.0
    dinv = lax.rsqrt(deg)
    y_ref[...] = jnp.dot(x_ref[...], w_ref[...],
                         preferred_element_type=jnp.float32) * dinv
    dinv_ref[...] = dinv


_tc_first = pl.pallas_call(
    _tc_first_body,
    grid=(_GRID,),
    in_specs=[
        pl.BlockSpec((_RB, F), lambda i: (i, 0)),
        pl.BlockSpec((_RB, 1), lambda i: (i, 0)),
        pl.BlockSpec((_RB, 1), lambda i: (i, 0)),
        pl.BlockSpec((F, F), lambda i: (0, 0)),
    ],
    out_specs=[
        pl.BlockSpec((_RB, F), lambda i: (i, 0)),
        pl.BlockSpec((_RB, 1), lambda i: (i, 0)),
    ],
    out_shape=[
        jax.ShapeDtypeStruct((N, F), jnp.float32),
        jax.ShapeDtypeStruct((N, 1), jnp.float32),
    ],
)


def _tc_mid_body(z0_ref, z1_ref, yp_ref, dinv_ref, b_ref, w_ref, y_ref):
    dinv = dinv_ref[...]
    h = (z0_ref[...] + z1_ref[...] - yp_ref[...]) * dinv + b_ref[...]
    h = jnp.maximum(h, 0.0)
    y_ref[...] = jnp.dot(h, w_ref[...],
                         preferred_element_type=jnp.float32) * dinv


_tc_mid = pl.pallas_call(
    _tc_mid_body,
    grid=(_GRID,),
    in_specs=[
        pl.BlockSpec((_RB, F), lambda i: (i, 0)),
        pl.BlockSpec((_RB, F), lambda i: (i, 0)),
        pl.BlockSpec((_RB, F), lambda i: (i, 0)),
        pl.BlockSpec((_RB, 1), lambda i: (i, 0)),
        pl.BlockSpec((1, F), lambda i: (0, 0)),
        pl.BlockSpec((F, F), lambda i: (0, 0)),
    ],
    out_specs=pl.BlockSpec((_RB, F), lambda i: (i, 0)),
    out_shape=jax.ShapeDtypeStruct((N, F), jnp.float32),
)


def _tc_final_body(z0_ref, z1_ref, yp_ref, dinv_ref, b_ref, out_ref):
    out_ref[...] = ((z0_ref[...] + z1_ref[...] - yp_ref[...])
                    * dinv_ref[...] + b_ref[...])


_tc_final = pl.pallas_call(
    _tc_final_body,
    grid=(_GRID,),
    in_specs=[
        pl.BlockSpec((_RB, F), lambda i: (i, 0)),
        pl.BlockSpec((_RB, F), lambda i: (i, 0)),
        pl.BlockSpec((_RB, F), lambda i: (i, 0)),
        pl.BlockSpec((_RB, 1), lambda i: (i, 0)),
        pl.BlockSpec((1, F), lambda i: (0, 0)),
    ],
    out_specs=pl.BlockSpec((_RB, F), lambda i: (i, 0)),
    out_shape=jax.ShapeDtypeStruct((N, F), jnp.float32),
)


def kernel(x, edge_index, W1, b1, Wh, bh, W2, b2):
    edge_index = edge_index.astype(jnp.int32)
    src4 = edge_index[0].reshape(NW, NCHUNK, CB, EB)
    dst4 = edge_index[1].reshape(NW, NCHUNK, CB, EB)
    dst3 = edge_index[1].reshape(NW, BLK_PER_TILE, EB)

    zeros_pad = jnp.zeros((NPAD, DEGW), jnp.float32)
    ones_blk = jnp.ones((EB, DEGW), jnp.float32)
    deg2 = _sc_degree(dst3, zeros_pad, ones_blk)
    deg_a = deg2[0, :N, 0].reshape(N, 1)
    deg_b = deg2[1, :N, 0].reshape(N, 1)

    y1, dinv = _tc_first(x, deg_a, deg_b, W1)
    z = _sc_aggregate(y1, src4, dst4)
    y2 = _tc_mid(z[0], z[1], y1, dinv, b1.reshape(1, F), Wh)
    z = _sc_aggregate(y2, src4, dst4)
    y3 = _tc_mid(z[0], z[1], y2, dinv, bh.reshape(1, F), W2)
    z = _sc_aggregate(y3, src4, dst4)
    return _tc_final(z[0], z[1], y3, dinv, b2.reshape(1, F))


# speedup vs baseline: 25.5834x; 1.3881x over previous
import functools

import jax
import jax.numpy as jnp
from jax import lax
from jax.experimental import pallas as pl
from jax.experimental.pallas import tpu as pltpu
from jax.experimental.pallas import tpu_sc as plsc

N = 10000
E = 320000
F = 128

NC = 2
NS = 16
NW = NC * NS
EB = 80
EROWS = E // EB
BLK_PER_TILE = EROWS // NW
NCHUNK = 5
CB = BLK_PER_TILE // NCHUNK
NPT = 624
NTAIL_OFF = NPT * NS
NTAIL = N - NTAIL_OFF

NPAD = 10240
DSEG = NPAD // NS
DEGW = 128

_mesh = plsc.VectorSubcoreMesh(
    core_axis_name="c", subcore_axis_name="s",
    num_cores=NC, num_subcores=NS)


@functools.partial(
    pl.kernel,
    out_type=jax.ShapeDtypeStruct((NC, NPAD, DEGW), jnp.float32),
    mesh=_mesh,
    scratch_types=[
        pltpu.VMEM((BLK_PER_TILE, EB), jnp.int32),
        pltpu.VMEM((EB, DEGW), jnp.float32),
        pltpu.VMEM_SHARED((NPAD, DEGW), jnp.float32),
    ],
)
def _sc_degree(dst_hbm, zeros_hbm, ones_hbm, out_hbm, dst_v, ones_v, deg_sh):
    c = lax.axis_index("c")
    s = lax.axis_index("s")
    wid = s * NC + c
    pltpu.sync_copy(zeros_hbm.at[pl.ds(s * DSEG, DSEG)],
                    deg_sh.at[pl.ds(s * DSEG, DSEG)])
    pltpu.sync_copy(ones_hbm, ones_v)
    pltpu.sync_copy(dst_hbm.at[wid], dst_v)
    plsc.subcore_barrier()

    def step(j, carry):
        pltpu.sync_copy(ones_v, deg_sh.at[dst_v.at[j]], add=True)
        return carry

    lax.fori_loop(0, BLK_PER_TILE, step, 0)
    plsc.subcore_barrier()
    pltpu.sync_copy(deg_sh.at[pl.ds(s * DSEG, DSEG)],
                    out_hbm.at[c, pl.ds(s * DSEG, DSEG)])


_RB = 1000
_GRID = N // _RB


def _tc_first_body(x_ref, da_ref, db_ref, w_ref, y_ref, dinv_ref):
    deg = da_ref[...] + db_ref[...] + 1.0
    dinv = lax.rsqrt(deg)
    y_ref[...] = jnp.dot(x_ref[...], w_ref[...],
                         preferred_element_type=jnp.float32) * dinv
    dinv_ref[...] = dinv


_tc_first = pl.pallas_call(
    _tc_first_body,
    grid=(_GRID,),
    in_specs=[
        pl.BlockSpec((_RB, F), lambda i: (i, 0)),
        pl.BlockSpec((_RB, 1), lambda i: (i, 0)),
        pl.BlockSpec((_RB, 1), lambda i: (i, 0)),
        pl.BlockSpec((F, F), lambda i: (0, 0)),
    ],
    out_specs=[
        pl.BlockSpec((_RB, F), lambda i: (i, 0)),
        pl.BlockSpec((_RB, 1), lambda i: (i, 0)),
    ],
    out_shape=[
        jax.ShapeDtypeStruct((N, F), jnp.float32),
        jax.ShapeDtypeStruct((N, 1), jnp.float32),
    ],
)


def _tc_mid_body(z0_ref, z1_ref, yp_ref, dinv_ref, b_ref, w_ref, y_ref):
    dinv = dinv_ref[...]
    h = (z0_ref[...] + z1_ref[...] - yp_ref[...]) * dinv + b_ref[...]
    h = jnp.maximum(h, 0.0)
    y_ref[...] = jnp.dot(h, w_ref[...],
                         preferred_element_type=jnp.float32) * dinv


_tc_mid = pl.pallas_call(
    _tc_mid_body,
    grid=(_GRID,),
    in_specs=[
        pl.BlockSpec((_RB, F), lambda i: (i, 0)),
        pl.BlockSpec((_RB, F), lambda i: (i, 0)),
        pl.BlockSpec((_RB, F), lambda i: (i, 0)),
        pl.BlockSpec((_RB, 1), lambda i: (i, 0)),
        pl.BlockSpec((1, F), lambda i: (0, 0)),
        pl.BlockSpec((F, F), lambda i: (0, 0)),
    ],
    out_specs=pl.BlockSpec((_RB, F), lambda i: (i, 0)),
    out_shape=jax.ShapeDtypeStruct((N, F), jnp.float32),
)


def _tc_final_body(z0_ref, z1_ref, yp_ref, dinv_ref, b_ref, out_ref):
    out_ref[...] = ((z0_ref[...] + z1_ref[...] - yp_ref[...])
                    * dinv_ref[...] + b_ref[...])


_tc_final = pl.pallas_call(
    _tc_final_body,
    grid=(_GRID,),
    in_specs=[
        pl.BlockSpec((_RB, F), lambda i: (i, 0)),
        pl.BlockSpec((_RB, F), lambda i: (i, 0)),
        pl.BlockSpec((_RB, F), lambda i: (i, 0)),
        pl.BlockSpec((_RB, 1), lambda i: (i, 0)),
        pl.BlockSpec((1, F), lambda i: (0, 0)),
    ],
    out_specs=pl.BlockSpec((_RB, F), lambda i: (i, 0)),
    out_shape=jax.ShapeDtypeStruct((N, F), jnp.float32),
)


def _make_agg(eb, depth, cb):
    blk = E // NW // eb
    nchunk = blk // cb
    assert blk % cb == 0 and depth - 1 <= cb

    @functools.partial(
        pl.kernel,
        out_type=jax.ShapeDtypeStruct((NC, N, F), jnp.float32),
        mesh=_mesh,
        scratch_types=[
            pltpu.VMEM((2, cb, eb), jnp.int32),
            pltpu.VMEM((2, cb, eb), jnp.int32),
            pltpu.VMEM((depth, eb, F), jnp.float32),
            pltpu.VMEM_SHARED((N, F), jnp.float32),
            pltpu.SemaphoreType.DMA((depth,)),
            pltpu.SemaphoreType.DMA((depth,)),
            pltpu.SemaphoreType.DMA((2,)),
        ],
    )
    def agg(y_hbm, src_hbm, dst_hbm, out_hbm,
            src_v, dst_v, rows_v, z_sh, gsem, ssem, isem):
        c = lax.axis_index("c")
        s = lax.axis_index("s")
        wid = s * NC + c
        pltpu.sync_copy(y_hbm.at[pl.ds(s * NPT, NPT)],
                        z_sh.at[pl.ds(s * NPT, NPT)])

        @pl.when(s == 0)
        def _init_tail():
            pltpu.sync_copy(y_hbm.at[pl.ds(NTAIL_OFF, NTAIL)],
                            z_sh.at[pl.ds(NTAIL_OFF, NTAIL)])

        pltpu.sync_copy(src_hbm.at[wid, 0], src_v.at[0])
        pltpu.sync_copy(dst_hbm.at[wid, 0], dst_v.at[0])
        plsc.subcore_barrier()

        for i in range(depth - 1):
            pltpu.async_copy(y_hbm.at[src_v.at[0, i]], rows_v.at[i],
                             gsem.at[i])

        def step(j, carry):
            b = lax.rem(j, depth)
            r = lax.rem(j, cb)
            kp = lax.rem(lax.div(j, cb), 2)
            pltpu.make_async_copy(y_hbm.at[src_v.at[kp, r]], rows_v.at[b],
                                  gsem.at[b]).wait()
            pltpu.async_copy(rows_v.at[b], z_sh.at[dst_v.at[kp, r]],
                             ssem.at[b], add=True)

            bp = lax.rem(j + depth - 1, depth)

            @pl.when(j > 0)
            def _drain():
                pltpu.make_async_copy(rows_v.at[bp], z_sh.at[dst_v.at[kp, r]],
                                      ssem.at[bp]).wait()

            @pl.when(jnp.logical_and(r == 0, j + cb < blk))
            def _stage_chunk():
                k = lax.div(j, cb)
                pltpu.async_copy(src_hbm.at[wid, k + 1], src_v.at[1 - kp],
                                 isem.at[0])
                pltpu.async_copy(dst_hbm.at[wid, k + 1], dst_v.at[1 - kp],
                                 isem.at[1])

            m = j + depth - 1

            @pl.when(m < blk)
            def _prefetch():
                rm = lax.rem(m, cb)
                kpm = lax.rem(lax.div(m, cb), 2)

                @pl.when(rm == 0)
                def _wait_chunk():
                    pltpu.make_async_copy(src_hbm.at[wid, 0], src_v.at[0],
                                          isem.at[0]).wait()
                    pltpu.make_async_copy(dst_hbm.at[wid, 0], dst_v.at[0],
                                          isem.at[1]).wait()

                pltpu.async_copy(y_hbm.at[src_v.at[kpm, rm]], rows_v.at[bp],
                                 gsem.at[bp])
            return carry

        lax.fori_loop(0, blk, step, 0)
        pltpu.make_async_copy(rows_v.at[lax.rem(blk - 1, depth)],
                              z_sh.at[dst_v.at[0, 0]],
                              ssem.at[lax.rem(blk - 1, depth)]).wait()
        plsc.subcore_barrier()
        pltpu.sync_copy(z_sh.at[pl.ds(s * NPT, NPT)],
                        out_hbm.at[c, pl.ds(s * NPT, NPT)])

        @pl.when(s == 0)
        def _out_tail():
            pltpu.sync_copy(z_sh.at[pl.ds(NTAIL_OFF, NTAIL)],
                            out_hbm.at[c, pl.ds(NTAIL_OFF, NTAIL)])
    return agg


_AGG_EB = 40
_AGG_DEPTH = 6
_AGG_CB = 25
_AGG_NCHUNK = E // NW // _AGG_EB // _AGG_CB
_sc_aggregate_v2 = _make_agg(_AGG_EB, _AGG_DEPTH, _AGG_CB)


def kernel(x, edge_index, W1, b1, Wh, bh, W2, b2):
    edge_index = edge_index.astype(jnp.int32)
    src4 = edge_index[0].reshape(NW, _AGG_NCHUNK, _AGG_CB, _AGG_EB)
    dst4 = edge_index[1].reshape(NW, _AGG_NCHUNK, _AGG_CB, _AGG_EB)
    dst3 = edge_index[1].reshape(NW, BLK_PER_TILE, EB)

    zeros_pad = jnp.zeros((NPAD, DEGW), jnp.float32)
    ones_blk = jnp.ones((EB, DEGW), jnp.float32)
    deg2 = _sc_degree(dst3, zeros_pad, ones_blk)
    deg_a = deg2[0, :N, 0].reshape(N, 1)
    deg_b = deg2[1, :N, 0].reshape(N, 1)

    y1, dinv = _tc_first(x, deg_a, deg_b, W1)
    z = _sc_aggregate_v2(y1, src4, dst4)
    y2 = _tc_mid(z[0], z[1], y1, dinv, b1.reshape(1, F), Wh)
    z = _sc_aggregate_v2(y2, src4, dst4)
    y3 = _tc_mid(z[0], z[1], y2, dinv, bh.reshape(1, F), W2)
    z = _sc_aggregate_v2(y3, src4, dst4)
    return _tc_final(z[0], z[1], y3, dinv, b2.reshape(1, F))


# speedup vs baseline: 29.4631x; 1.1516x over previous
import functools

import jax
import jax.numpy as jnp
from jax import lax
from jax.experimental import pallas as pl
from jax.experimental.pallas import tpu as pltpu
from jax.experimental.pallas import tpu_sc as plsc

N = 10000
E = 320000
F = 128

NC = 2
NS = 16
NW = NC * NS
EB = 80
EROWS = E // EB
BLK_PER_TILE = EROWS // NW
NCHUNK = 5
CB = BLK_PER_TILE // NCHUNK
NPT = 624
NTAIL_OFF = NPT * NS
NTAIL = N - NTAIL_OFF

NPAD = 10240
DSEG = NPAD // NS
DEGW = 128

_mesh = plsc.VectorSubcoreMesh(
    core_axis_name="c", subcore_axis_name="s",
    num_cores=NC, num_subcores=NS)


@functools.partial(
    pl.kernel,
    out_type=jax.ShapeDtypeStruct((NC, NPAD), jnp.float32),
    mesh=_mesh,
    compiler_params=pltpu.CompilerParams(needs_layout_passes=False),
    scratch_types=[
        pltpu.VMEM((BLK_PER_TILE, EB), jnp.int32),
        pltpu.VMEM((NPAD,), jnp.float32),
        pltpu.VMEM((NS, DSEG), jnp.float32),
        pltpu.VMEM_SHARED((NS, NPAD), jnp.float32),
    ],
)
def _sc_degree(dst_hbm, out_hbm, dst_v, hist_v, red_v, sh):
    c = lax.axis_index("c")
    s = lax.axis_index("s")
    wid = s * NC + c
    pltpu.sync_copy(dst_hbm.at[wid], dst_v)

    def zero(i, carry):
        hist_v[pl.ds(i * 16, 16)] = jnp.zeros((16,), jnp.float32)
        return carry

    lax.fori_loop(0, NPAD // 16, zero, 0)
    ones = jnp.ones((16,), jnp.float32)

    def accum(i, carry):
        for g in range(EB // 16):
            idx = dst_v[i, pl.ds(g * 16, 16)]
            plsc.addupdate_scatter(hist_v, [idx], ones)
        return carry

    lax.fori_loop(0, BLK_PER_TILE, accum, 0)
    pltpu.sync_copy(hist_v, sh.at[s])
    plsc.subcore_barrier()
    pltpu.sync_copy(sh.at[:, pl.ds(s * DSEG, DSEG)], red_v)

    def red(i, carry):
        acc = jnp.zeros((16,), jnp.float32)
        for r in range(NS):
            acc = acc + red_v[r, pl.ds(i * 16, 16)]
        hist_v[pl.ds(i * 16, 16)] = acc
        return carry

    lax.fori_loop(0, DSEG // 16, red, 0)
    pltpu.sync_copy(hist_v.at[pl.ds(0, DSEG)],
                    out_hbm.at[c, pl.ds(s * DSEG, DSEG)])


_RB = 1000
_GRID = N // _RB


def _tc_first_body(x_ref, da_ref, db_ref, w_ref, y_ref, dinv_ref):
    deg = da_ref[...] + db_ref[...] + 1.0
    dinv = lax.rsqrt(deg)
    y_ref[...] = jnp.dot(x_ref[...], w_ref[...],
                         preferred_element_type=jnp.float32) * dinv
    dinv_ref[...] = dinv


_tc_first = pl.pallas_call(
    _tc_first_body,
    grid=(_GRID,),
    in_specs=[
        pl.BlockSpec((_RB, F), lambda i: (i, 0)),
        pl.BlockSpec((_RB, 1), lambda i: (i, 0)),
        pl.BlockSpec((_RB, 1), lambda i: (i, 0)),
        pl.BlockSpec((F, F), lambda i: (0, 0)),
    ],
    out_specs=[
        pl.BlockSpec((_RB, F), lambda i: (i, 0)),
        pl.BlockSpec((_RB, 1), lambda i: (i, 0)),
    ],
    out_shape=[
        jax.ShapeDtypeStruct((N, F), jnp.float32),
        jax.ShapeDtypeStruct((N, 1), jnp.float32),
    ],
)


def _tc_mid_body(z0_ref, z1_ref, yp_ref, dinv_ref, b_ref, w_ref, y_ref):
    dinv = dinv_ref[...]
    h = (z0_ref[...] + z1_ref[...] - yp_ref[...]) * dinv + b_ref[...]
    h = jnp.maximum(h, 0.0)
    y_ref[...] = jnp.dot(h, w_ref[...],
                         preferred_element_type=jnp.float32) * dinv


_tc_mid = pl.pallas_call(
    _tc_mid_body,
    grid=(_GRID,),
    in_specs=[
        pl.BlockSpec((_RB, F), lambda i: (i, 0)),
        pl.BlockSpec((_RB, F), lambda i: (i, 0)),
        pl.BlockSpec((_RB, F), lambda i: (i, 0)),
        pl.BlockSpec((_RB, 1), lambda i: (i, 0)),
        pl.BlockSpec((1, F), lambda i: (0, 0)),
        pl.BlockSpec((F, F), lambda i: (0, 0)),
    ],
    out_specs=pl.BlockSpec((_RB, F), lambda i: (i, 0)),
    out_shape=jax.ShapeDtypeStruct((N, F), jnp.float32),
)


def _tc_final_body(z0_ref, z1_ref, yp_ref, dinv_ref, b_ref, out_ref):
    out_ref[...] = ((z0_ref[...] + z1_ref[...] - yp_ref[...])
                    * dinv_ref[...] + b_ref[...])


_tc_final = pl.pallas_call(
    _tc_final_body,
    grid=(_GRID,),
    in_specs=[
        pl.BlockSpec((_RB, F), lambda i: (i, 0)),
        pl.BlockSpec((_RB, F), lambda i: (i, 0)),
        pl.BlockSpec((_RB, F), lambda i: (i, 0)),
        pl.BlockSpec((_RB, 1), lambda i: (i, 0)),
        pl.BlockSpec((1, F), lambda i: (0, 0)),
    ],
    out_specs=pl.BlockSpec((_RB, F), lambda i: (i, 0)),
    out_shape=jax.ShapeDtypeStruct((N, F), jnp.float32),
)


def _make_agg(eb, depth, cb):
    blk = E // NW // eb
    nchunk = blk // cb
    assert blk % cb == 0 and depth - 1 <= cb

    @functools.partial(
        pl.kernel,
        out_type=jax.ShapeDtypeStruct((NC, N, F), jnp.float32),
        mesh=_mesh,
        scratch_types=[
            pltpu.VMEM((2, cb, eb), jnp.int32),
            pltpu.VMEM((2, cb, eb), jnp.int32),
            pltpu.VMEM((depth, eb, F), jnp.float32),
            pltpu.VMEM_SHARED((N, F), jnp.float32),
            pltpu.SemaphoreType.DMA((depth,)),
            pltpu.SemaphoreType.DMA((depth,)),
            pltpu.SemaphoreType.DMA((2,)),
        ],
    )
    def agg(y_hbm, src_hbm, dst_hbm, out_hbm,
            src_v, dst_v, rows_v, z_sh, gsem, ssem, isem):
        c = lax.axis_index("c")
        s = lax.axis_index("s")
        wid = s * NC + c
        pltpu.sync_copy(y_hbm.at[pl.ds(s * NPT, NPT)],
                        z_sh.at[pl.ds(s * NPT, NPT)])

        @pl.when(s == 0)
        def _init_tail():
            pltpu.sync_copy(y_hbm.at[pl.ds(NTAIL_OFF, NTAIL)],
                            z_sh.at[pl.ds(NTAIL_OFF, NTAIL)])

        pltpu.sync_copy(src_hbm.at[wid, 0], src_v.at[0])
        pltpu.sync_copy(dst_hbm.at[wid, 0], dst_v.at[0])
        plsc.subcore_barrier()

        for i in range(depth - 1):
            pltpu.async_copy(y_hbm.at[src_v.at[0, i]], rows_v.at[i],
                             gsem.at[i])

        def step(j, carry):
            b = lax.rem(j, depth)
            r = lax.rem(j, cb)
            kp = lax.rem(lax.div(j, cb), 2)
            pltpu.make_async_copy(y_hbm.at[src_v.at[kp, r]], rows_v.at[b],
                                  gsem.at[b]).wait()
            pltpu.async_copy(rows_v.at[b], z_sh.at[dst_v.at[kp, r]],
                             ssem.at[b], add=True)

            bp = lax.rem(j + depth - 1, depth)

            @pl.when(j > 0)
            def _drain():
                pltpu.make_async_copy(rows_v.at[bp], z_sh.at[dst_v.at[kp, r]],
                                      ssem.at[bp]).wait()

            @pl.when(jnp.logical_and(r == 0, j + cb < blk))
            def _stage_chunk():
                k = lax.div(j, cb)
                pltpu.async_copy(src_hbm.at[wid, k + 1], src_v.at[1 - kp],
                                 isem.at[0])
                pltpu.async_copy(dst_hbm.at[wid, k + 1], dst_v.at[1 - kp],
                                 isem.at[1])

            m = j + depth - 1

            @pl.when(m < blk)
            def _prefetch():
                rm = lax.rem(m, cb)
                kpm = lax.rem(lax.div(m, cb), 2)

                @pl.when(rm == 0)
                def _wait_chunk():
                    pltpu.make_async_copy(src_hbm.at[wid, 0], src_v.at[0],
                                          isem.at[0]).wait()
                    pltpu.make_async_copy(dst_hbm.at[wid, 0], dst_v.at[0],
                                          isem.at[1]).wait()

                pltpu.async_copy(y_hbm.at[src_v.at[kpm, rm]], rows_v.at[bp],
                                 gsem.at[bp])
            return carry

        lax.fori_loop(0, blk, step, 0)
        pltpu.make_async_copy(rows_v.at[lax.rem(blk - 1, depth)],
                              z_sh.at[dst_v.at[0, 0]],
                              ssem.at[lax.rem(blk - 1, depth)]).wait()
        plsc.subcore_barrier()
        pltpu.sync_copy(z_sh.at[pl.ds(s * NPT, NPT)],
                        out_hbm.at[c, pl.ds(s * NPT, NPT)])

        @pl.when(s == 0)
        def _out_tail():
            pltpu.sync_copy(z_sh.at[pl.ds(NTAIL_OFF, NTAIL)],
                            out_hbm.at[c, pl.ds(NTAIL_OFF, NTAIL)])
    return agg


_AGG_EB = 40
_AGG_DEPTH = 6
_AGG_CB = 25
_AGG_NCHUNK = E // NW // _AGG_EB // _AGG_CB
_sc_aggregate_v2 = _make_agg(_AGG_EB, _AGG_DEPTH, _AGG_CB)


def kernel(x, edge_index, W1, b1, Wh, bh, W2, b2):
    edge_index = edge_index.astype(jnp.int32)
    src4 = edge_index[0].reshape(NW, _AGG_NCHUNK, _AGG_CB, _AGG_EB)
    dst4 = edge_index[1].reshape(NW, _AGG_NCHUNK, _AGG_CB, _AGG_EB)
    dst3 = edge_index[1].reshape(NW, BLK_PER_TILE, EB)

    deg2 = _sc_degree(dst3)
    deg_a = deg2[0, :N].reshape(N, 1)
    deg_b = deg2[1, :N].reshape(N, 1)

    y1, dinv = _tc_first(x, deg_a, deg_b, W1)
    z = _sc_aggregate_v2(y1, src4, dst4)
    y2 = _tc_mid(z[0], z[1], y1, dinv, b1.reshape(1, F), Wh)
    z = _sc_aggregate_v2(y2, src4, dst4)
    y3 = _tc_mid(z[0], z[1], y2, dinv, bh.reshape(1, F), W2)
    z = _sc_aggregate_v2(y3, src4, dst4)
    return _tc_final(z[0], z[1], y3, dinv, b2.reshape(1, F))


# speedup vs baseline: 30.6409x; 1.0400x over previous
import functools

import jax
import jax.numpy as jnp
from jax import lax
from jax.experimental import pallas as pl
from jax.experimental.pallas import tpu as pltpu
from jax.experimental.pallas import tpu_sc as plsc

N = 10000
E = 320000
F = 128

NC = 2
NS = 16
NW = NC * NS
EB = 80
EROWS = E // EB
BLK_PER_TILE = EROWS // NW
NCHUNK = 5
CB = BLK_PER_TILE // NCHUNK
NPT = 624
NTAIL_OFF = NPT * NS
NTAIL = N - NTAIL_OFF

NPAD = 10240
DSEG = NPAD // NS
DEGW = 128

_mesh = plsc.VectorSubcoreMesh(
    core_axis_name="c", subcore_axis_name="s",
    num_cores=NC, num_subcores=NS)


@functools.partial(
    pl.kernel,
    out_type=jax.ShapeDtypeStruct((NC, NPAD), jnp.float32),
    mesh=_mesh,
    compiler_params=pltpu.CompilerParams(needs_layout_passes=False),
    scratch_types=[
        pltpu.VMEM((BLK_PER_TILE, EB), jnp.int32),
        pltpu.VMEM((NPAD,), jnp.float32),
        pltpu.VMEM((NS, DSEG), jnp.float32),
        pltpu.VMEM_SHARED((NS, NPAD), jnp.float32),
    ],
)
def _sc_degree(dst_hbm, out_hbm, dst_v, hist_v, red_v, sh):
    c = lax.axis_index("c")
    s = lax.axis_index("s")
    wid = s * NC + c
    pltpu.sync_copy(dst_hbm.at[wid], dst_v)

    def zero(i, carry):
        hist_v[pl.ds(i * 16, 16)] = jnp.zeros((16,), jnp.float32)
        return carry

    lax.fori_loop(0, NPAD // 16, zero, 0)
    ones = jnp.ones((16,), jnp.float32)

    def accum(i, carry):
        for g in range(EB // 16):
            idx = dst_v[i, pl.ds(g * 16, 16)]
            plsc.addupdate_scatter(hist_v, [idx], ones)
        return carry

    lax.fori_loop(0, BLK_PER_TILE, accum, 0)
    pltpu.sync_copy(hist_v, sh.at[s])
    plsc.subcore_barrier()
    pltpu.sync_copy(sh.at[:, pl.ds(s * DSEG, DSEG)], red_v)

    def red(i, carry):
        acc = jnp.zeros((16,), jnp.float32)
        for r in range(NS):
            acc = acc + red_v[r, pl.ds(i * 16, 16)]
        hist_v[pl.ds(i * 16, 16)] = acc
        return carry

    lax.fori_loop(0, DSEG // 16, red, 0)
    pltpu.sync_copy(hist_v.at[pl.ds(0, DSEG)],
                    out_hbm.at[c, pl.ds(s * DSEG, DSEG)])


_RB = 1000
_GRID = N // _RB


def _tc_first_body(x_ref, da_ref, db_ref, w_ref, y_ref, dinv_ref):
    deg = da_ref[...] + db_ref[...] + 1.0
    dinv = lax.rsqrt(deg)
    y_ref[...] = jnp.dot(x_ref[...], w_ref[...],
                         preferred_element_type=jnp.float32) * dinv
    dinv_ref[...] = dinv


_tc_first = pl.pallas_call(
    _tc_first_body,
    grid=(_GRID,),
    in_specs=[
        pl.BlockSpec((_RB, F), lambda i: (i, 0)),
        pl.BlockSpec((_RB, 1), lambda i: (i, 0)),
        pl.BlockSpec((_RB, 1), lambda i: (i, 0)),
        pl.BlockSpec((F, F), lambda i: (0, 0)),
    ],
    out_specs=[
        pl.BlockSpec((_RB, F), lambda i: (i, 0)),
        pl.BlockSpec((_RB, 1), lambda i: (i, 0)),
    ],
    out_shape=[
        jax.ShapeDtypeStruct((N, F), jnp.float32),
        jax.ShapeDtypeStruct((N, 1), jnp.float32),
    ],
)


def _tc_mid_body(z0_ref, z1_ref, yp_ref, dinv_ref, b_ref, w_ref, y_ref):
    dinv = dinv_ref[...]
    h = (z0_ref[...] + z1_ref[...] - yp_ref[...]) * dinv + b_ref[...]
    h = jnp.maximum(h, 0.0)
    y_ref[...] = jnp.dot(h, w_ref[...],
                         preferred_element_type=jnp.float32) * dinv


_tc_mid = pl.pallas_call(
    _tc_mid_body,
    grid=(_GRID,),
    in_specs=[
        pl.BlockSpec((_RB, F), lambda i: (i, 0)),
        pl.BlockSpec((_RB, F), lambda i: (i, 0)),
        pl.BlockSpec((_RB, F), lambda i: (i, 0)),
        pl.BlockSpec((_RB, 1), lambda i: (i, 0)),
        pl.BlockSpec((1, F), lambda i: (0, 0)),
        pl.BlockSpec((F, F), lambda i: (0, 0)),
    ],
    out_specs=pl.BlockSpec((_RB, F), lambda i: (i, 0)),
    out_shape=jax.ShapeDtypeStruct((N, F), jnp.float32),
)


def _tc_final_body(z0_ref, z1_ref, yp_ref, dinv_ref, b_ref, out_ref):
    out_ref[...] = ((z0_ref[...] + z1_ref[...] - yp_ref[...])
                    * dinv_ref[...] + b_ref[...])


_tc_final = pl.pallas_call(
    _tc_final_body,
    grid=(_GRID,),
    in_specs=[
        pl.BlockSpec((_RB, F), lambda i: (i, 0)),
        pl.BlockSpec((_RB, F), lambda i: (i, 0)),
        pl.BlockSpec((_RB, F), lambda i: (i, 0)),
        pl.BlockSpec((_RB, 1), lambda i: (i, 0)),
        pl.BlockSpec((1, F), lambda i: (0, 0)),
    ],
    out_specs=pl.BlockSpec((_RB, F), lambda i: (i, 0)),
    out_shape=jax.ShapeDtypeStruct((N, F), jnp.float32),
)


def _make_agg(eb, depth, cb):
    blk = E // NW // eb
    nchunk = blk // cb
    assert blk % cb == 0 and depth - 1 <= cb

    @functools.partial(
        pl.kernel,
        out_type=jax.ShapeDtypeStruct((NC, N, F), jnp.float32),
        mesh=_mesh,
        scratch_types=[
            pltpu.VMEM((2, cb, eb), jnp.int32),
            pltpu.VMEM((2, cb, eb), jnp.int32),
            pltpu.VMEM((depth, eb, F), jnp.float32),
            pltpu.VMEM_SHARED((N, F), jnp.float32),
            pltpu.SemaphoreType.DMA((depth,)),
            pltpu.SemaphoreType.DMA((depth,)),
            pltpu.SemaphoreType.DMA((2,)),
            pltpu.SemaphoreType.DMA,
        ],
    )
    def agg(y_hbm, src_hbm, dst_hbm, out_hbm,
            src_v, dst_v, rows_v, z_sh, gsem, ssem, isem, psem):
        c = lax.axis_index("c")
        s = lax.axis_index("s")
        wid = s * NC + c
        pltpu.async_copy(y_hbm.at[pl.ds(s * NPT, NPT)],
                         z_sh.at[pl.ds(s * NPT, NPT)], psem)

        @pl.when(s == 0)
        def _init_tail():
            pltpu.async_copy(y_hbm.at[pl.ds(NTAIL_OFF, NTAIL)],
                             z_sh.at[pl.ds(NTAIL_OFF, NTAIL)], psem)

        pltpu.async_copy(src_hbm.at[wid, 0], src_v.at[0], isem.at[0])
        pltpu.async_copy(dst_hbm.at[wid, 0], dst_v.at[0], isem.at[1])
        pltpu.make_async_copy(src_hbm.at[wid, 0], src_v.at[0],
                              isem.at[0]).wait()
        pltpu.make_async_copy(dst_hbm.at[wid, 0], dst_v.at[0],
                              isem.at[1]).wait()

        for i in range(depth - 1):
            pltpu.async_copy(y_hbm.at[src_v.at[0, i]], rows_v.at[i],
                             gsem.at[i])

        pltpu.make_async_copy(y_hbm.at[pl.ds(s * NPT, NPT)],
                              z_sh.at[pl.ds(s * NPT, NPT)], psem).wait()

        @pl.when(s == 0)
        def _init_tail_wait():
            pltpu.make_async_copy(y_hbm.at[pl.ds(NTAIL_OFF, NTAIL)],
                                  z_sh.at[pl.ds(NTAIL_OFF, NTAIL)],
                                  psem).wait()

        plsc.subcore_barrier()

        def step(j, carry):
            b = lax.rem(j, depth)
            r = lax.rem(j, cb)
            kp = lax.rem(lax.div(j, cb), 2)
            pltpu.make_async_copy(y_hbm.at[src_v.at[kp, r]], rows_v.at[b],
                                  gsem.at[b]).wait()
            pltpu.async_copy(rows_v.at[b], z_sh.at[dst_v.at[kp, r]],
                             ssem.at[b], add=True)

            bp = lax.rem(j + depth - 1, depth)

            @pl.when(j > 0)
            def _drain():
                pltpu.make_async_copy(rows_v.at[bp], z_sh.at[dst_v.at[kp, r]],
                                      ssem.at[bp]).wait()

            @pl.when(jnp.logical_and(r == 0, j + cb < blk))
            def _stage_chunk():
                k = lax.div(j, cb)
                pltpu.async_copy(src_hbm.at[wid, k + 1], src_v.at[1 - kp],
                                 isem.at[0])
                pltpu.async_copy(dst_hbm.at[wid, k + 1], dst_v.at[1 - kp],
                                 isem.at[1])

            m = j + depth - 1

            @pl.when(m < blk)
            def _prefetch():
                rm = lax.rem(m, cb)
                kpm = lax.rem(lax.div(m, cb), 2)

                @pl.when(rm == 0)
                def _wait_chunk():
                    pltpu.make_async_copy(src_hbm.at[wid, 0], src_v.at[0],
                                          isem.at[0]).wait()
                    pltpu.make_async_copy(dst_hbm.at[wid, 0], dst_v.at[0],
                                          isem.at[1]).wait()

                pltpu.async_copy(y_hbm.at[src_v.at[kpm, rm]], rows_v.at[bp],
                                 gsem.at[bp])
            return carry

        lax.fori_loop(0, blk, step, 0)
        pltpu.make_async_copy(rows_v.at[lax.rem(blk - 1, depth)],
                              z_sh.at[dst_v.at[0, 0]],
                              ssem.at[lax.rem(blk - 1, depth)]).wait()
        plsc.subcore_barrier()
        pltpu.sync_copy(z_sh.at[pl.ds(s * NPT, NPT)],
                        out_hbm.at[c, pl.ds(s * NPT, NPT)])

        @pl.when(s == 0)
        def _out_tail():
            pltpu.sync_copy(z_sh.at[pl.ds(NTAIL_OFF, NTAIL)],
                            out_hbm.at[c, pl.ds(NTAIL_OFF, NTAIL)])
    return agg


_AGG_EB = 40
_AGG_DEPTH = 7
_AGG_CB = 10
_AGG_NCHUNK = E // NW // _AGG_EB // _AGG_CB
_sc_aggregate_v2 = _make_agg(_AGG_EB, _AGG_DEPTH, _AGG_CB)


def kernel(x, edge_index, W1, b1, Wh, bh, W2, b2):
    edge_index = edge_index.astype(jnp.int32)
    src4 = edge_index[0].reshape(NW, _AGG_NCHUNK, _AGG_CB, _AGG_EB)
    dst4 = edge_index[1].reshape(NW, _AGG_NCHUNK, _AGG_CB, _AGG_EB)
    dst3 = edge_index[1].reshape(NW, BLK_PER_TILE, EB)

    deg2 = _sc_degree(dst3)
    deg_a = deg2[0, :N].reshape(N, 1)
    deg_b = deg2[1, :N].reshape(N, 1)

    y1, dinv = _tc_first(x, deg_a, deg_b, W1)
    z = _sc_aggregate_v2(y1, src4, dst4)
    y2 = _tc_mid(z[0], z[1], y1, dinv, b1.reshape(1, F), Wh)
    z = _sc_aggregate_v2(y2, src4, dst4)
    y3 = _tc_mid(z[0], z[1], y2, dinv, bh.reshape(1, F), W2)
    z = _sc_aggregate_v2(y3, src4, dst4)
    return _tc_final(z[0], z[1], y3, dinv, b2.reshape(1, F))


# speedup vs baseline: 31.2919x; 1.0212x over previous
import functools

import jax
import jax.numpy as jnp
from jax import lax
from jax.experimental import pallas as pl
from jax.experimental.pallas import tpu as pltpu
from jax.experimental.pallas import tpu_sc as plsc

N = 10000
E = 320000
F = 128

NC = 2
NS = 16
NW = NC * NS
EB = 80
EROWS = E // EB
BLK_PER_TILE = EROWS // NW
NCHUNK = 5
CB = BLK_PER_TILE // NCHUNK
NPT = 624
NTAIL_OFF = NPT * NS
NTAIL = N - NTAIL_OFF

NPAD = 10240
DSEG = NPAD // NS
DEGW = 128

_mesh = plsc.VectorSubcoreMesh(
    core_axis_name="c", subcore_axis_name="s",
    num_cores=NC, num_subcores=NS)


@functools.partial(
    pl.kernel,
    out_type=jax.ShapeDtypeStruct((NC, NPAD), jnp.float32),
    mesh=_mesh,
    compiler_params=pltpu.CompilerParams(needs_layout_passes=False),
    scratch_types=[
        pltpu.VMEM((BLK_PER_TILE, EB), jnp.int32),
        pltpu.VMEM((NPAD,), jnp.float32),
        pltpu.VMEM((NS, DSEG), jnp.float32),
        pltpu.VMEM_SHARED((NS, NPAD), jnp.float32),
    ],
)
def _sc_degree(dst_hbm, out_hbm, dst_v, hist_v, red_v, sh):
    c = lax.axis_index("c")
    s = lax.axis_index("s")
    wid = s * NC + c
    pltpu.sync_copy(dst_hbm.at[wid], dst_v)

    def zero(i, carry):
        hist_v[pl.ds(i * 16, 16)] = jnp.zeros((16,), jnp.float32)
        return carry

    lax.fori_loop(0, NPAD // 16, zero, 0)
    ones = jnp.ones((16,), jnp.float32)

    def accum(i, carry):
        for g in range(EB // 16):
            idx = dst_v[i, pl.ds(g * 16, 16)]
            plsc.addupdate_scatter(hist_v, [idx], ones)
        return carry

    lax.fori_loop(0, BLK_PER_TILE, accum, 0)
    pltpu.sync_copy(hist_v, sh.at[s])
    plsc.subcore_barrier()
    pltpu.sync_copy(sh.at[:, pl.ds(s * DSEG, DSEG)], red_v)

    def red(i, carry):
        acc = jnp.zeros((16,), jnp.float32)
        for r in range(NS):
            acc = acc + red_v[r, pl.ds(i * 16, 16)]
        hist_v[pl.ds(i * 16, 16)] = acc
        return carry

    lax.fori_loop(0, DSEG // 16, red, 0)
    pltpu.sync_copy(hist_v.at[pl.ds(0, DSEG)],
                    out_hbm.at[c, pl.ds(s * DSEG, DSEG)])


_RB = 2000
_GRID = N // _RB


def _tc_first_body(x_ref, da_ref, db_ref, w_ref, y_ref, dinv_ref):
    deg = da_ref[...] + db_ref[...] + 1.0
    dinv = lax.rsqrt(deg)
    y_ref[...] = jnp.dot(x_ref[...], w_ref[...],
                         preferred_element_type=jnp.float32) * dinv
    dinv_ref[...] = dinv


_tc_first = pl.pallas_call(
    _tc_first_body,
    grid=(_GRID,),
    in_specs=[
        pl.BlockSpec((_RB, F), lambda i: (i, 0)),
        pl.BlockSpec((_RB, 1), lambda i: (i, 0)),
        pl.BlockSpec((_RB, 1), lambda i: (i, 0)),
        pl.BlockSpec((F, F), lambda i: (0, 0)),
    ],
    out_specs=[
        pl.BlockSpec((_RB, F), lambda i: (i, 0)),
        pl.BlockSpec((_RB, 1), lambda i: (i, 0)),
    ],
    out_shape=[
        jax.ShapeDtypeStruct((N, F), jnp.float32),
        jax.ShapeDtypeStruct((N, 1), jnp.float32),
    ],
)


def _tc_mid_body(z0_ref, z1_ref, yp_ref, dinv_ref, b_ref, w_ref, y_ref):
    dinv = dinv_ref[...]
    h = (z0_ref[...] + z1_ref[...] - yp_ref[...]) * dinv + b_ref[...]
    h = jnp.maximum(h, 0.0)
    y_ref[...] = jnp.dot(h, w_ref[...],
                         preferred_element_type=jnp.float32) * dinv


_tc_mid = pl.pallas_call(
    _tc_mid_body,
    grid=(_GRID,),
    in_specs=[
        pl.BlockSpec((_RB, F), lambda i: (i, 0)),
        pl.BlockSpec((_RB, F), lambda i: (i, 0)),
        pl.BlockSpec((_RB, F), lambda i: (i, 0)),
        pl.BlockSpec((_RB, 1), lambda i: (i, 0)),
        pl.BlockSpec((1, F), lambda i: (0, 0)),
        pl.BlockSpec((F, F), lambda i: (0, 0)),
    ],
    out_specs=pl.BlockSpec((_RB, F), lambda i: (i, 0)),
    out_shape=jax.ShapeDtypeStruct((N, F), jnp.float32),
)


def _tc_final_body(z0_ref, z1_ref, yp_ref, dinv_ref, b_ref, out_ref):
    out_ref[...] = ((z0_ref[...] + z1_ref[...] - yp_ref[...])
                    * dinv_ref[...] + b_ref[...])


_tc_final = pl.pallas_call(
    _tc_final_body,
    grid=(_GRID,),
    in_specs=[
        pl.BlockSpec((_RB, F), lambda i: (i, 0)),
        pl.BlockSpec((_RB, F), lambda i: (i, 0)),
        pl.BlockSpec((_RB, F), lambda i: (i, 0)),
        pl.BlockSpec((_RB, 1), lambda i: (i, 0)),
        pl.BlockSpec((1, F), lambda i: (0, 0)),
    ],
    out_specs=pl.BlockSpec((_RB, F), lambda i: (i, 0)),
    out_shape=jax.ShapeDtypeStruct((N, F), jnp.float32),
)


def _make_agg(eb, depth, cb):
    blk = E // NW // eb
    nchunk = blk // cb
    assert blk % cb == 0 and depth - 1 <= cb

    @functools.partial(
        pl.kernel,
        out_type=jax.ShapeDtypeStruct((NC, N, F), jnp.float32),
        mesh=_mesh,
        scratch_types=[
            pltpu.VMEM((2, cb, eb), jnp.int32),
            pltpu.VMEM((2, cb, eb), jnp.int32),
            pltpu.VMEM((depth, eb, F), jnp.float32),
            pltpu.VMEM_SHARED((N, F), jnp.float32),
            pltpu.SemaphoreType.DMA((depth,)),
            pltpu.SemaphoreType.DMA((depth,)),
            pltpu.SemaphoreType.DMA((2,)),
            pltpu.SemaphoreType.DMA,
        ],
    )
    def agg(y_hbm, src_hbm, dst_hbm, out_hbm,
            src_v, dst_v, rows_v, z_sh, gsem, ssem, isem, psem):
        c = lax.axis_index("c")
        s = lax.axis_index("s")
        wid = s * NC + c
        pltpu.async_copy(y_hbm.at[pl.ds(s * NPT, NPT)],
                         z_sh.at[pl.ds(s * NPT, NPT)], psem)

        @pl.when(s == 0)
        def _init_tail():
            pltpu.async_copy(y_hbm.at[pl.ds(NTAIL_OFF, NTAIL)],
                             z_sh.at[pl.ds(NTAIL_OFF, NTAIL)], psem)

        pltpu.async_copy(src_hbm.at[wid, 0], src_v.at[0], isem.at[0])
        pltpu.async_copy(dst_hbm.at[wid, 0], dst_v.at[0], isem.at[1])
        pltpu.make_async_copy(src_hbm.at[wid, 0], src_v.at[0],
                              isem.at[0]).wait()
        pltpu.make_async_copy(dst_hbm.at[wid, 0], dst_v.at[0],
                              isem.at[1]).wait()

        for i in range(depth - 1):
            pltpu.async_copy(y_hbm.at[src_v.at[0, i]], rows_v.at[i],
                             gsem.at[i])

        pltpu.make_async_copy(y_hbm.at[pl.ds(s * NPT, NPT)],
                              z_sh.at[pl.ds(s * NPT, NPT)], psem).wait()

        @pl.when(s == 0)
        def _init_tail_wait():
            pltpu.make_async_copy(y_hbm.at[pl.ds(NTAIL_OFF, NTAIL)],
                                  z_sh.at[pl.ds(NTAIL_OFF, NTAIL)],
                                  psem).wait()

        plsc.subcore_barrier()

        def step(j, carry):
            b = lax.rem(j, depth)
            r = lax.rem(j, cb)
            kp = lax.rem(lax.div(j, cb), 2)
            pltpu.make_async_copy(y_hbm.at[src_v.at[kp, r]], rows_v.at[b],
                                  gsem.at[b]).wait()
            pltpu.async_copy(rows_v.at[b], z_sh.at[dst_v.at[kp, r]],
                             ssem.at[b], add=True)

            bp = lax.rem(j + depth - 1, depth)

            @pl.when(j > 0)
            def _drain():
                pltpu.make_async_copy(rows_v.at[bp], z_sh.at[dst_v.at[kp, r]],
                                      ssem.at[bp]).wait()

            @pl.when(jnp.logical_and(r == 0, j + cb < blk))
            def _stage_chunk():
                k = lax.div(j, cb)
                pltpu.async_copy(src_hbm.at[wid, k + 1], src_v.at[1 - kp],
                                 isem.at[0])
                pltpu.async_copy(dst_hbm.at[wid, k + 1], dst_v.at[1 - kp],
                                 isem.at[1])

            m = j + depth - 1

            @pl.when(m < blk)
            def _prefetch():
                rm = lax.rem(m, cb)
                kpm = lax.rem(lax.div(m, cb), 2)

                @pl.when(rm == 0)
                def _wait_chunk():
                    pltpu.make_async_copy(src_hbm.at[wid, 0], src_v.at[0],
                                          isem.at[0]).wait()
                    pltpu.make_async_copy(dst_hbm.at[wid, 0], dst_v.at[0],
                                          isem.at[1]).wait()

                pltpu.async_copy(y_hbm.at[src_v.at[kpm, rm]], rows_v.at[bp],
                                 gsem.at[bp])
            return carry

        lax.fori_loop(0, blk, step, 0)
        pltpu.make_async_copy(rows_v.at[lax.rem(blk - 1, depth)],
                              z_sh.at[dst_v.at[0, 0]],
                              ssem.at[lax.rem(blk - 1, depth)]).wait()
        plsc.subcore_barrier()
        pltpu.sync_copy(z_sh.at[pl.ds(s * NPT, NPT)],
                        out_hbm.at[c, pl.ds(s * NPT, NPT)])

        @pl.when(s == 0)
        def _out_tail():
            pltpu.sync_copy(z_sh.at[pl.ds(NTAIL_OFF, NTAIL)],
                            out_hbm.at[c, pl.ds(NTAIL_OFF, NTAIL)])
    return agg


_AGG_EB = 40
_AGG_DEPTH = 7
_AGG_CB = 10
_AGG_NCHUNK = E // NW // _AGG_EB // _AGG_CB
_sc_aggregate_v2 = _make_agg(_AGG_EB, _AGG_DEPTH, _AGG_CB)


def kernel(x, edge_index, W1, b1, Wh, bh, W2, b2):
    edge_index = edge_index.astype(jnp.int32)
    src4 = edge_index[0].reshape(NW, _AGG_NCHUNK, _AGG_CB, _AGG_EB)
    dst4 = edge_index[1].reshape(NW, _AGG_NCHUNK, _AGG_CB, _AGG_EB)
    dst3 = edge_index[1].reshape(NW, BLK_PER_TILE, EB)

    deg2 = _sc_degree(dst3)
    deg_a = deg2[0, :N].reshape(N, 1)
    deg_b = deg2[1, :N].reshape(N, 1)

    y1, dinv = _tc_first(x, deg_a, deg_b, W1)
    z = _sc_aggregate_v2(y1, src4, dst4)
    y2 = _tc_mid(z[0], z[1], y1, dinv, b1.reshape(1, F), Wh)
    z = _sc_aggregate_v2(y2, src4, dst4)
    y3 = _tc_mid(z[0], z[1], y2, dinv, bh.reshape(1, F), W2)
    z = _sc_aggregate_v2(y3, src4, dst4)
    return _tc_final(z[0], z[1], y3, dinv, b2.reshape(1, F))
